# baseline (speedup 1.0000x reference)
import jax
import jax.numpy as jnp
from jax import lax
from jax.experimental import pallas as pl
from jax.experimental.pallas import tpu as pltpu
from jax.experimental.pallas import tpu_sc as plsc

_MAXLEN = 8192
_EMBED = 128
_BATCH = 4
_NC = 2
_NS = 16
_ROWS = _MAXLEN // (_NC * _NS)
_CHUNK = 128
_HALVES = _ROWS // _CHUNK
_NCHUNKS = _BATCH * _HALVES
_RING = 4
_LANES = 16


def _sc_body(x_hbm, pos_hbm, out_hbm, pos_v, ring0, ring1, ring2, ring3,
             *sems):
    wid = lax.axis_index("s") * _NC + lax.axis_index("c")
    t0 = wid * _ROWS

    ring = (ring0, ring1, ring2, ring3)
    lsems = sems[0:_RING]
    ssems = sems[_RING:2 * _RING]

    loads, stores = {}, {}
    loads[0] = pltpu.async_copy(
        x_hbm.at[0, pl.ds(t0, _CHUNK)], ring[0], lsems[0])
    pltpu.sync_copy(pos_hbm.at[pl.ds(t0, _ROWS)], pos_v)

    for step in range(_NCHUNKS):
        if step + 1 < _NCHUNKS:
            c = step + 1
            if c >= _RING:
                stores[c - _RING].wait()
            b, h = divmod(c, _HALVES)
            loads[c] = pltpu.async_copy(
                x_hbm.at[b, pl.ds(t0 + h * _CHUNK, _CHUNK)],
                ring[c % _RING], lsems[c % _RING])
        c = step
        loads[c].wait()
        buf = ring[c % _RING]
        b, h = divmod(c, _HALVES)
        poff = h * _CHUNK

        @plsc.parallel_loop(0, _CHUNK, step=1, unroll=4)
        def _row(r):
            for k in range(_EMBED // _LANES):
                sl = pl.ds(k * _LANES, _LANES)
                buf[r, sl] = buf[r, sl] + pos_v[poff + r, sl]

        stores[c] = pltpu.async_copy(
            buf, out_hbm.at[b, pl.ds(t0 + h * _CHUNK, _CHUNK)],
            ssems[c % _RING])
    for c in range(_NCHUNKS - _RING, _NCHUNKS):
        stores[c].wait()


def kernel(x, pos_table):
    mesh = plsc.VectorSubcoreMesh(core_axis_name="c", subcore_axis_name="s",
                                  num_cores=_NC, num_subcores=_NS)
    run = pl.kernel(
        _sc_body,
        out_type=jax.ShapeDtypeStruct((_BATCH, _MAXLEN, _EMBED), jnp.float32),
        mesh=mesh,
        scratch_types=(
            [pltpu.VMEM((_ROWS, _EMBED), jnp.float32)]
            + [pltpu.VMEM((_CHUNK, _EMBED), jnp.float32)] * _RING
            + [pltpu.SemaphoreType.DMA] * (2 * _RING)
        ),
    )
    return run(x, pos_table)


# speedup vs baseline: 2.7571x; 2.7571x over previous
import jax
import jax.numpy as jnp
from jax.experimental import pallas as pl

_MAXLEN = 8192
_EMBED = 128
_BATCH = 4
_TBLK = 8192


def _add_body(x_ref, p_ref, o_ref):
    o_ref[...] = x_ref[...] + p_ref[...]


def kernel(x, pos_table):
    grid = (_MAXLEN // _TBLK, _BATCH)
    return pl.pallas_call(
        _add_body,
        grid=grid,
        in_specs=[
            pl.BlockSpec((1, _TBLK, _EMBED), lambda t, b: (b, t, 0)),
            pl.BlockSpec((_TBLK, _EMBED), lambda t, b: (t, 0)),
        ],
        out_specs=pl.BlockSpec((1, _TBLK, _EMBED), lambda t, b: (b, t, 0)),
        out_shape=jax.ShapeDtypeStruct((_BATCH, _MAXLEN, _EMBED), jnp.float32),
    )(x, pos_table)


# speedup vs baseline: 3.1110x; 1.1284x over previous
import jax
import jax.numpy as jnp
from jax.experimental import pallas as pl

_MAXLEN = 8192
_EMBED = 128
_BATCH = 4
_BBLK = 2


def _add_body(x_ref, p_ref, o_ref):
    o_ref[...] = x_ref[...] + p_ref[...][None, :, :]


def kernel(x, pos_table):
    return pl.pallas_call(
        _add_body,
        grid=(_BATCH // _BBLK,),
        in_specs=[
            pl.BlockSpec((_BBLK, _MAXLEN, _EMBED), lambda b: (b, 0, 0)),
            pl.BlockSpec((_MAXLEN, _EMBED), lambda b: (0, 0)),
        ],
        out_specs=pl.BlockSpec((_BBLK, _MAXLEN, _EMBED), lambda b: (b, 0, 0)),
        out_shape=jax.ShapeDtypeStruct((_BATCH, _MAXLEN, _EMBED), jnp.float32),
    )(x, pos_table)
